# baseline (speedup 1.0000x reference)
import jax
import jax.numpy as jnp
from jax.experimental import pallas as pl
from jax.experimental.pallas import tpu as pltpu

_HI = jax.lax.Precision.HIGHEST


def _iota2(shape, dim):
    return jax.lax.broadcasted_iota(jnp.int32, shape, dim)


def _pool_mats(n_sub, n_lane):
    st = jnp.where(_iota2((n_sub // 4, n_sub), 1) // 4
                   == _iota2((n_sub // 4, n_sub), 0), 1.0, 0.0)
    m = jnp.where(_iota2((n_lane, n_lane // 4), 0) // 4
                  == _iota2((n_lane, n_lane // 4), 1), 1.0, 0.0)
    return st, m


def _pool_fine_to_mid(x_ref, o_ref, copy_ref):
    xs = x_ref[0] + x_ref[1] + x_ref[2] + x_ref[3]
    st, m = _pool_mats(256, 64)
    t = jnp.dot(xs, m, precision=_HI)
    o_ref[0] = jnp.dot(st, t, precision=_HI) * (1.0 / 64.0)
    copy_ref[...] = x_ref[...]


def _count(pred):
    return jnp.sum(pred.astype(jnp.int32))


def _order_key(v):
    s = jax.lax.bitcast_convert_type(v, jnp.int32)
    return s ^ ((s >> 31) & jnp.int32(0x7FFFFFFF))


def _bottomk_mask(key, idx, k, n):
    c_neg = _count(key < 0)
    neg = k <= c_neg
    lo = jnp.where(neg, jnp.int32(-2147483648), jnp.int32(0))
    hi = jnp.where(neg, jnp.int32(-1), jnp.int32(2147483647))

    def body(_, c):
        lo, hi = c
        mid = lo + (hi - lo) // 2
        ge = _count(key <= mid) >= k
        return jnp.where(ge, lo, mid + 1), jnp.where(ge, mid, hi)

    lo, hi = jax.lax.fori_loop(0, 31, body, (lo, hi))
    t = lo
    c_lt = _count(key < t)
    need = k - c_lt

    def body2(_, c):
        lo, hi = c
        mid = lo + (hi - lo) // 2
        ge = _count((key == t) & (idx < mid)) >= need
        return jnp.where(ge, lo, mid + 1), jnp.where(ge, mid, hi)

    cut, _unused = jax.lax.fori_loop(0, 17, body2,
                                     (jnp.int32(0), jnp.int32(n)))
    mask = ((key < t) | ((key == t) & (idx < cut))).astype(jnp.int32)
    mask = jnp.where(k >= n, jnp.int32(1), mask)
    mask = jnp.where(k <= 0, jnp.int32(0), mask)
    return mask


def _flat_idx(shape):
    d0, d1, d2 = shape
    return (_iota3(shape, 0) * (d1 * d2) + _iota3(shape, 1) * d2
            + _iota3(shape, 2))


def _iota3(shape, dim):
    return jax.lax.broadcasted_iota(jnp.int32, shape, dim)


def _select(k_ref, dmid_ref, dc_ref, m0_ref, m1_ref):
    st, m = _pool_mats(64, 16)
    rows = []
    for i in range(16):
        t = (dmid_ref[4 * i] + dmid_ref[4 * i + 1]
             + dmid_ref[4 * i + 2] + dmid_ref[4 * i + 3])
        c = jnp.dot(st, jnp.dot(t, m, precision=_HI),
                    precision=_HI) * (1.0 / 64.0)
        rows.append(c[None])
    c3 = jnp.concatenate(rows, axis=0)
    dc_ref[...] = c3

    kc = _order_key(c3)
    km = _order_key(dmid_ref[...])
    m0 = _bottomk_mask(kc, _flat_idx((16, 16, 4)), k_ref[0], 1024)
    m1 = _bottomk_mask(km, _flat_idx((64, 64, 16)), k_ref[1], 65536)
    m0_ref[...] = m0
    m1_ref[...] = m1


def _up_mats(n_from_sub, n_from_lane):
    w = jnp.where(_iota2((4 * n_from_sub, n_from_sub), 0) // 4
                  == _iota2((4 * n_from_sub, n_from_sub), 1), 1.0, 0.0)
    u = jnp.where(_iota2((n_from_lane, 4 * n_from_lane), 1) // 4
                  == _iota2((n_from_lane, 4 * n_from_lane), 0), 1.0, 0.0)
    return w, u


def _combine(m0_ref, m1_ref, sf_ref, m2_ref, c1_ref):
    w1, u1 = _up_mats(16, 4)
    s_c = jnp.where(m0_ref[0] != 0, 0.0, 2.0)
    s_up = jnp.dot(jnp.dot(w1, s_c, precision=_HI), u1,
                   precision=_HI)
    w2, u2 = _up_mats(64, 16)
    for r in range(4):
        m1r = m1_ref[r] != 0
        cond1 = m1r & (s_up > 0.0)
        c1_ref[r] = cond1.astype(jnp.int8)
        s_mid = jnp.where(cond1, 1.0, s_up)
        sf2d = jnp.dot(jnp.dot(w2, s_mid, precision=_HI), u2,
                       precision=_HI)
        sf_ref[pl.ds(4 * r, 4)] = jnp.broadcast_to(
            sf2d[None], (4, 256, 64))
        m2_ref[pl.ds(4 * r, 4)] = jnp.broadcast_to(
            (sf2d > 1.0)[None], (4, 256, 64)).astype(jnp.int8)


def kernel(density, selected_scale_index, selected_grain_ratio):
    del selected_scale_index
    density = density.astype(jnp.float32)
    ks = jnp.floor(
        selected_grain_ratio[:2].astype(jnp.float32)
        * jnp.array([1024.0, 65536.0], jnp.float32)).astype(jnp.int32)

    d_mid, d_fine = pl.pallas_call(
        _pool_fine_to_mid,
        grid=(64,),
        in_specs=[pl.BlockSpec((4, 256, 64), lambda g: (g, 0, 0))],
        out_specs=[
            pl.BlockSpec((1, 64, 16), lambda g: (g, 0, 0)),
            pl.BlockSpec((4, 256, 64), lambda g: (g, 0, 0)),
        ],
        out_shape=[
            jax.ShapeDtypeStruct((64, 64, 16), jnp.float32),
            jax.ShapeDtypeStruct((256, 256, 64), jnp.float32),
        ],
    )(density)

    d_coarse, m0, m1 = pl.pallas_call(
        _select,
        in_specs=[
            pl.BlockSpec(memory_space=pltpu.SMEM),
            pl.BlockSpec((64, 64, 16), lambda: (0, 0, 0)),
        ],
        out_specs=[
            pl.BlockSpec((16, 16, 4), lambda: (0, 0, 0)),
            pl.BlockSpec((16, 16, 4), lambda: (0, 0, 0)),
            pl.BlockSpec((64, 64, 16), lambda: (0, 0, 0)),
        ],
        out_shape=[
            jax.ShapeDtypeStruct((16, 16, 4), jnp.float32),
            jax.ShapeDtypeStruct((16, 16, 4), jnp.int32),
            jax.ShapeDtypeStruct((64, 64, 16), jnp.int32),
        ],
    )(ks, d_mid)

    s_fine, m2, c1 = pl.pallas_call(
        _combine,
        grid=(16,),
        in_specs=[
            pl.BlockSpec((1, 16, 4), lambda g: (g, 0, 0)),
            pl.BlockSpec((4, 64, 16), lambda g: (g, 0, 0)),
        ],
        out_specs=[
            pl.BlockSpec((16, 256, 64), lambda g: (g, 0, 0)),
            pl.BlockSpec((16, 256, 64), lambda g: (g, 0, 0)),
            pl.BlockSpec((4, 64, 16), lambda g: (g, 0, 0)),
        ],
        out_shape=[
            jax.ShapeDtypeStruct((256, 256, 64), jnp.float32),
            jax.ShapeDtypeStruct((256, 256, 64), jnp.int8),
            jax.ShapeDtypeStruct((64, 64, 16), jnp.int8),
        ],
    )(m0, m1)

    return (
        s_fine,
        m2.astype(jnp.bool_)[None, None],
        c1.astype(jnp.bool_)[None, None],
        (m0 != 0)[None, None],
        d_coarse[None, None],
        d_mid[None, None],
        d_fine[None, None],
    )


# speedup vs baseline: 1.1257x; 1.1257x over previous
import jax
import jax.numpy as jnp
from jax.experimental import pallas as pl
from jax.experimental.pallas import tpu as pltpu

_HI = jax.lax.Precision.HIGHEST


def _iota2(shape, dim):
    return jax.lax.broadcasted_iota(jnp.int32, shape, dim)


def _pool_mats(n_sub, n_lane):
    st = jnp.where(_iota2((n_sub // 4, n_sub), 1) // 4
                   == _iota2((n_sub // 4, n_sub), 0), 1.0, 0.0)
    m = jnp.where(_iota2((n_lane, n_lane // 4), 0) // 4
                  == _iota2((n_lane, n_lane // 4), 1), 1.0, 0.0)
    return st, m


def _pool_fine_to_mid(x_ref, o_ref):
    st, m = _pool_mats(256, 64)
    for i in range(8):
        xs = (x_ref[4 * i] + x_ref[4 * i + 1]
              + x_ref[4 * i + 2] + x_ref[4 * i + 3])
        t = jnp.dot(xs, m, precision=_HI)
        o_ref[i] = jnp.dot(st, t, precision=_HI) * (1.0 / 64.0)


def _count(pred):
    return jnp.sum(pred.astype(jnp.int32))


def _order_key(v):
    s = jax.lax.bitcast_convert_type(v, jnp.int32)
    return s ^ ((s >> 31) & jnp.int32(0x7FFFFFFF))


def _bottomk_mask(key, idx, k, n):
    c_neg = _count(key < 0)
    neg = k <= c_neg
    lo = jnp.where(neg, jnp.int32(-2147483648), jnp.int32(0))
    hi = jnp.where(neg, jnp.int32(-1), jnp.int32(2147483647))

    def body(_, c):
        lo, hi = c
        mid = lo + (hi - lo) // 2
        ge = _count(key <= mid) >= k
        return jnp.where(ge, lo, mid + 1), jnp.where(ge, mid, hi)

    lo, hi = jax.lax.fori_loop(0, 31, body, (lo, hi))
    t = lo
    c_lt = _count(key < t)
    need = k - c_lt

    def body2(_, c):
        lo, hi = c
        mid = lo + (hi - lo) // 2
        ge = _count((key == t) & (idx < mid)) >= need
        return jnp.where(ge, lo, mid + 1), jnp.where(ge, mid, hi)

    cut, _unused = jax.lax.fori_loop(0, 17, body2,
                                     (jnp.int32(0), jnp.int32(n)))
    mask = ((key < t) | ((key == t) & (idx < cut))).astype(jnp.int32)
    mask = jnp.where(k >= n, jnp.int32(1), mask)
    mask = jnp.where(k <= 0, jnp.int32(0), mask)
    return mask


def _flat_idx(shape):
    d0, d1, d2 = shape
    return (_iota3(shape, 0) * (d1 * d2) + _iota3(shape, 1) * d2
            + _iota3(shape, 2))


def _iota3(shape, dim):
    return jax.lax.broadcasted_iota(jnp.int32, shape, dim)


def _select(k_ref, dmid_ref, dc_ref, m0_ref, m1_ref):
    st, m = _pool_mats(64, 16)
    rows = []
    for i in range(16):
        t = (dmid_ref[4 * i] + dmid_ref[4 * i + 1]
             + dmid_ref[4 * i + 2] + dmid_ref[4 * i + 3])
        c = jnp.dot(st, jnp.dot(t, m, precision=_HI),
                    precision=_HI) * (1.0 / 64.0)
        rows.append(c[None])
    c3 = jnp.concatenate(rows, axis=0)
    dc_ref[...] = c3

    kc = _order_key(c3)
    km = _order_key(dmid_ref[...])
    m0 = _bottomk_mask(kc, _flat_idx((16, 16, 4)), k_ref[0], 1024)
    m1 = _bottomk_mask(km, _flat_idx((64, 64, 16)), k_ref[1], 65536)
    m0_ref[...] = m0
    m1_ref[...] = m1


def _up_mats(n_from_sub, n_from_lane):
    w = jnp.where(_iota2((4 * n_from_sub, n_from_sub), 0) // 4
                  == _iota2((4 * n_from_sub, n_from_sub), 1), 1.0, 0.0)
    u = jnp.where(_iota2((n_from_lane, 4 * n_from_lane), 1) // 4
                  == _iota2((n_from_lane, 4 * n_from_lane), 0), 1.0, 0.0)
    return w, u


def _combine(m0_ref, m1_ref, sf_ref, m2_ref, c1_ref):
    w1, u1 = _up_mats(16, 4)
    s_c = jnp.where(m0_ref[0] != 0, 0.0, 2.0)
    s_up = jnp.dot(jnp.dot(w1, s_c, precision=_HI), u1,
                   precision=_HI)
    w2, u2 = _up_mats(64, 16)
    for r in range(4):
        m1r = m1_ref[r] != 0
        cond1 = m1r & (s_up > 0.0)
        c1_ref[r] = cond1
        s_mid = jnp.where(cond1, 1.0, s_up)
        sf2d = jnp.dot(jnp.dot(w2, s_mid, precision=_HI), u2,
                       precision=_HI)
        sf_ref[pl.ds(4 * r, 4)] = jnp.broadcast_to(
            sf2d[None], (4, 256, 64))
        m2_ref[pl.ds(4 * r, 4)] = jnp.broadcast_to(
            (sf2d > 1.0)[None], (4, 256, 64))


def kernel(density, selected_scale_index, selected_grain_ratio):
    del selected_scale_index
    density = density.astype(jnp.float32)
    ks = jnp.floor(
        selected_grain_ratio[:2].astype(jnp.float32)
        * jnp.array([1024.0, 65536.0], jnp.float32)).astype(jnp.int32)

    d_mid = pl.pallas_call(
        _pool_fine_to_mid,
        grid=(8,),
        in_specs=[pl.BlockSpec((32, 256, 64), lambda g: (g, 0, 0))],
        out_specs=pl.BlockSpec((8, 64, 16), lambda g: (g, 0, 0)),
        out_shape=jax.ShapeDtypeStruct((64, 64, 16), jnp.float32),
        compiler_params=pltpu.CompilerParams(
            dimension_semantics=("parallel",)),
    )(density)
    d_fine = density

    d_coarse, m0, m1 = pl.pallas_call(
        _select,
        in_specs=[
            pl.BlockSpec(memory_space=pltpu.SMEM),
            pl.BlockSpec((64, 64, 16), lambda: (0, 0, 0)),
        ],
        out_specs=[
            pl.BlockSpec((16, 16, 4), lambda: (0, 0, 0)),
            pl.BlockSpec((16, 16, 4), lambda: (0, 0, 0)),
            pl.BlockSpec((64, 64, 16), lambda: (0, 0, 0)),
        ],
        out_shape=[
            jax.ShapeDtypeStruct((16, 16, 4), jnp.float32),
            jax.ShapeDtypeStruct((16, 16, 4), jnp.int32),
            jax.ShapeDtypeStruct((64, 64, 16), jnp.int32),
        ],
    )(ks, d_mid)

    s_fine, m2, c1 = pl.pallas_call(
        _combine,
        grid=(16,),
        in_specs=[
            pl.BlockSpec((1, 16, 4), lambda g: (g, 0, 0)),
            pl.BlockSpec((4, 64, 16), lambda g: (g, 0, 0)),
        ],
        out_specs=[
            pl.BlockSpec((16, 256, 64), lambda g: (g, 0, 0)),
            pl.BlockSpec((16, 256, 64), lambda g: (g, 0, 0)),
            pl.BlockSpec((4, 64, 16), lambda g: (g, 0, 0)),
        ],
        out_shape=[
            jax.ShapeDtypeStruct((256, 256, 64), jnp.float32),
            jax.ShapeDtypeStruct((256, 256, 64), jnp.bool_),
            jax.ShapeDtypeStruct((64, 64, 16), jnp.bool_),
        ],
        compiler_params=pltpu.CompilerParams(
            dimension_semantics=("parallel",)),
    )(m0, m1)

    return (
        s_fine,
        m2[None, None],
        c1[None, None],
        (m0 != 0)[None, None],
        d_coarse[None, None],
        d_mid[None, None],
        d_fine[None, None],
    )


# speedup vs baseline: 1.1881x; 1.0555x over previous
import jax
import jax.numpy as jnp
from jax.experimental import pallas as pl
from jax.experimental.pallas import tpu as pltpu

_HI = jax.lax.Precision.HIGHEST


def _iota2(shape, dim):
    return jax.lax.broadcasted_iota(jnp.int32, shape, dim)


def _pool_mats(n_sub, n_lane):
    st = jnp.where(_iota2((n_sub // 4, n_sub), 1) // 4
                   == _iota2((n_sub // 4, n_sub), 0), 1.0, 0.0)
    m = jnp.where(_iota2((n_lane, n_lane // 4), 0) // 4
                  == _iota2((n_lane, n_lane // 4), 1), 1.0, 0.0)
    return st, m


def _pool_fine_to_mid(x_ref, o_ref):
    st, m = _pool_mats(256, 64)
    for i in range(8):
        xs = (x_ref[4 * i] + x_ref[4 * i + 1]
              + x_ref[4 * i + 2] + x_ref[4 * i + 3])
        t = jnp.dot(xs, m, precision=_HI)
        o_ref[i] = jnp.dot(st, t, precision=_HI) * (1.0 / 64.0)


def _count(pred):
    return jnp.sum(pred.astype(jnp.int32))


def _order_key(v):
    s = jax.lax.bitcast_convert_type(v, jnp.int32)
    return s ^ ((s >> 31) & jnp.int32(0x7FFFFFFF))


def _bottomk_mask(key, idx, k, n):
    c_neg = _count(key < 0)
    neg = k <= c_neg
    lo = jnp.where(neg, jnp.int32(-2147483648), jnp.int32(0))
    hi = jnp.where(neg, jnp.int32(-1), jnp.int32(2147483647))

    def body(_, c):
        lo, hi = c
        mid = lo + (hi - lo) // 2
        ge = _count(key <= mid) >= k
        return jnp.where(ge, lo, mid + 1), jnp.where(ge, mid, hi)

    lo, hi = jax.lax.fori_loop(0, 31, body, (lo, hi))
    t = lo
    c_lt = _count(key < t)
    need = k - c_lt

    def body2(_, c):
        lo, hi = c
        mid = lo + (hi - lo) // 2
        ge = _count((key == t) & (idx < mid)) >= need
        return jnp.where(ge, lo, mid + 1), jnp.where(ge, mid, hi)

    cut, _unused = jax.lax.fori_loop(0, 17, body2,
                                     (jnp.int32(0), jnp.int32(n)))
    mask = ((key < t) | ((key == t) & (idx < cut))).astype(jnp.int32)
    mask = jnp.where(k >= n, jnp.int32(1), mask)
    mask = jnp.where(k <= 0, jnp.int32(0), mask)
    return mask


def _flat_idx(shape):
    d0, d1, d2 = shape
    return (_iota3(shape, 0) * (d1 * d2) + _iota3(shape, 1) * d2
            + _iota3(shape, 2))


def _iota3(shape, dim):
    return jax.lax.broadcasted_iota(jnp.int32, shape, dim)


def _select(k_ref, dmid_ref, dc_ref, m0_ref, m1_ref):
    st, m = _pool_mats(64, 16)
    rows = []
    for i in range(16):
        t = (dmid_ref[4 * i] + dmid_ref[4 * i + 1]
             + dmid_ref[4 * i + 2] + dmid_ref[4 * i + 3])
        c = jnp.dot(st, jnp.dot(t, m, precision=_HI),
                    precision=_HI) * (1.0 / 64.0)
        rows.append(c[None])
    c3 = jnp.concatenate(rows, axis=0)
    dc_ref[...] = c3

    kc = _order_key(c3)
    km = _order_key(dmid_ref[...])
    m0 = _bottomk_mask(kc, _flat_idx((16, 16, 4)), k_ref[0], 1024)
    m1 = _bottomk_mask(km, _flat_idx((64, 64, 16)), k_ref[1], 65536)
    m0_ref[...] = m0
    m1_ref[...] = m1


def _up_mats(n_from_sub, n_from_lane):
    w = jnp.where(_iota2((4 * n_from_sub, n_from_sub), 0) // 4
                  == _iota2((4 * n_from_sub, n_from_sub), 1), 1.0, 0.0)
    u = jnp.where(_iota2((n_from_lane, 4 * n_from_lane), 1) // 4
                  == _iota2((n_from_lane, 4 * n_from_lane), 0), 1.0, 0.0)
    return w, u


def _combine(m0_ref, m1_ref, sf_ref, m2_ref, c1_ref):
    w1, u1 = _up_mats(16, 4)
    w2, u2 = _up_mats(64, 16)
    for q in range(2):
        s_c = jnp.where(m0_ref[q] != 0, 0.0, 2.0)
        s_up = jnp.dot(jnp.dot(w1, s_c), u1)
        for r in range(4):
            m1r = m1_ref[4 * q + r] != 0
            cond1 = m1r & (s_up > 0.0)
            c1_ref[0, 0, 4 * q + r] = cond1
            s_mid = jnp.where(cond1, 1.0, s_up)
            sf2d = jnp.dot(jnp.dot(w2, s_mid), u2)
            sf_ref[pl.ds(16 * q + 4 * r, 4)] = jnp.broadcast_to(
                sf2d[None], (4, 256, 64))
            m2_ref[0, 0, pl.ds(16 * q + 4 * r, 4)] = jnp.broadcast_to(
                (sf2d > 1.0)[None], (4, 256, 64))


def kernel(density, selected_scale_index, selected_grain_ratio):
    del selected_scale_index
    density = density.astype(jnp.float32)
    ks = jnp.floor(
        selected_grain_ratio[:2].astype(jnp.float32)
        * jnp.array([1024.0, 65536.0], jnp.float32)).astype(jnp.int32)

    d_mid = pl.pallas_call(
        _pool_fine_to_mid,
        grid=(8,),
        in_specs=[pl.BlockSpec((32, 256, 64), lambda g: (g, 0, 0))],
        out_specs=pl.BlockSpec((8, 64, 16), lambda g: (g, 0, 0)),
        out_shape=jax.ShapeDtypeStruct((64, 64, 16), jnp.float32),
        compiler_params=pltpu.CompilerParams(
            dimension_semantics=("parallel",)),
    )(density)
    d_fine = density

    d_coarse, m0, m1 = pl.pallas_call(
        _select,
        in_specs=[
            pl.BlockSpec(memory_space=pltpu.SMEM),
            pl.BlockSpec((64, 64, 16), lambda: (0, 0, 0)),
        ],
        out_specs=[
            pl.BlockSpec((16, 16, 4), lambda: (0, 0, 0)),
            pl.BlockSpec((16, 16, 4), lambda: (0, 0, 0)),
            pl.BlockSpec((64, 64, 16), lambda: (0, 0, 0)),
        ],
        out_shape=[
            jax.ShapeDtypeStruct((16, 16, 4), jnp.float32),
            jax.ShapeDtypeStruct((16, 16, 4), jnp.int32),
            jax.ShapeDtypeStruct((64, 64, 16), jnp.int32),
        ],
    )(ks, d_mid)

    s_fine, m2, c1 = pl.pallas_call(
        _combine,
        grid=(8,),
        in_specs=[
            pl.BlockSpec((2, 16, 4), lambda g: (g, 0, 0)),
            pl.BlockSpec((8, 64, 16), lambda g: (g, 0, 0)),
        ],
        out_specs=[
            pl.BlockSpec((32, 256, 64), lambda g: (g, 0, 0)),
            pl.BlockSpec((1, 1, 32, 256, 64), lambda g: (0, 0, g, 0, 0)),
            pl.BlockSpec((1, 1, 8, 64, 16), lambda g: (0, 0, g, 0, 0)),
        ],
        out_shape=[
            jax.ShapeDtypeStruct((256, 256, 64), jnp.float32),
            jax.ShapeDtypeStruct((1, 1, 256, 256, 64), jnp.bool_),
            jax.ShapeDtypeStruct((1, 1, 64, 64, 16), jnp.bool_),
        ],
        compiler_params=pltpu.CompilerParams(
            dimension_semantics=("parallel",)),
    )(m0, m1)

    return (
        s_fine,
        m2,
        c1,
        (m0 != 0)[None, None],
        d_coarse[None, None],
        d_mid[None, None],
        d_fine[None, None],
    )


# speedup vs baseline: 1.2297x; 1.0350x over previous
import jax
import jax.numpy as jnp
from jax.experimental import pallas as pl
from jax.experimental.pallas import tpu as pltpu

_HI = jax.lax.Precision.HIGHEST


def _iota2(shape, dim):
    return jax.lax.broadcasted_iota(jnp.int32, shape, dim)


def _pool_mats(n_sub, n_lane):
    st = jnp.where(_iota2((n_sub // 4, n_sub), 1) // 4
                   == _iota2((n_sub // 4, n_sub), 0), 1.0, 0.0)
    m = jnp.where(_iota2((n_lane, n_lane // 4), 0) // 4
                  == _iota2((n_lane, n_lane // 4), 1), 1.0, 0.0)
    return st, m


def _pool_fine_to_mid(x_ref, o_ref):
    st, m = _pool_mats(256, 64)
    for i in range(8):
        xs = (x_ref[4 * i] + x_ref[4 * i + 1]
              + x_ref[4 * i + 2] + x_ref[4 * i + 3])
        t = jnp.dot(xs, m, precision=_HI)
        o_ref[i] = jnp.dot(st, t, precision=_HI) * (1.0 / 64.0)


def _count(pred):
    return jnp.sum(pred.astype(jnp.int32))


def _order_key(v):
    s = jax.lax.bitcast_convert_type(v, jnp.int32)
    return s ^ ((s >> 31) & jnp.int32(0x7FFFFFFF))


def _bottomk_masks(keys, idxs, ks, ns):
    lo, hi = [], []
    for key, k in zip(keys, ks):
        c_neg = _count(key < 0)
        neg = k <= c_neg
        lo.append(jnp.where(neg, jnp.int32(-2147483648), jnp.int32(0)))
        hi.append(jnp.where(neg, jnp.int32(-1), jnp.int32(2147483647)))

    def body(_, c):
        l0, h0, l1, h1 = c
        m0 = l0 + (h0 - l0) // 2
        m1 = l1 + (h1 - l1) // 2
        g0 = _count(keys[0] <= m0) >= ks[0]
        g1 = _count(keys[1] <= m1) >= ks[1]
        return (jnp.where(g0, l0, m0 + 1), jnp.where(g0, m0, h0),
                jnp.where(g1, l1, m1 + 1), jnp.where(g1, m1, h1))

    t0, _h0, t1, _h1 = jax.lax.fori_loop(
        0, 31, body, (lo[0], hi[0], lo[1], hi[1]))
    ts = (t0, t1)
    needs = [k - _count(key < t) for key, k, t in zip(keys, ks, ts)]

    def body2(_, c):
        l0, h0, l1, h1 = c
        m0 = l0 + (h0 - l0) // 2
        m1 = l1 + (h1 - l1) // 2
        g0 = _count((keys[0] == ts[0]) & (idxs[0] < m0)) >= needs[0]
        g1 = _count((keys[1] == ts[1]) & (idxs[1] < m1)) >= needs[1]
        return (jnp.where(g0, l0, m0 + 1), jnp.where(g0, m0, h0),
                jnp.where(g1, l1, m1 + 1), jnp.where(g1, m1, h1))

    c0, _x0, c1, _x1 = jax.lax.fori_loop(
        0, 17, body2, (jnp.int32(0), jnp.int32(ns[0]),
                       jnp.int32(0), jnp.int32(ns[1])))
    masks = []
    for key, idx, k, n, t, cut in zip(keys, idxs, ks, ns, ts, (c0, c1)):
        mask = ((key < t) | ((key == t) & (idx < cut))).astype(jnp.int32)
        mask = jnp.where(k >= n, jnp.int32(1), mask)
        mask = jnp.where(k <= 0, jnp.int32(0), mask)
        masks.append(mask)
    return masks


def _flat_idx(shape):
    d0, d1, d2 = shape
    return (_iota3(shape, 0) * (d1 * d2) + _iota3(shape, 1) * d2
            + _iota3(shape, 2))


def _iota3(shape, dim):
    return jax.lax.broadcasted_iota(jnp.int32, shape, dim)


def _select(k_ref, dmid_ref, dc_ref, m0_ref, m1_ref):
    st, m = _pool_mats(64, 16)
    rows = []
    for i in range(16):
        t = (dmid_ref[4 * i] + dmid_ref[4 * i + 1]
             + dmid_ref[4 * i + 2] + dmid_ref[4 * i + 3])
        c = jnp.dot(st, jnp.dot(t, m, precision=_HI),
                    precision=_HI) * (1.0 / 64.0)
        rows.append(c[None])
    c3 = jnp.concatenate(rows, axis=0)
    dc_ref[...] = c3

    kc = _order_key(c3)
    km = _order_key(dmid_ref[...])
    m0, m1 = _bottomk_masks(
        (kc, km), (_flat_idx((16, 16, 4)), _flat_idx((64, 64, 16))),
        (k_ref[0], k_ref[1]), (1024, 65536))
    m0_ref[...] = m0
    m1_ref[...] = m1


def _up_mats(n_from_sub, n_from_lane):
    w = jnp.where(_iota2((4 * n_from_sub, n_from_sub), 0) // 4
                  == _iota2((4 * n_from_sub, n_from_sub), 1), 1.0, 0.0)
    u = jnp.where(_iota2((n_from_lane, 4 * n_from_lane), 1) // 4
                  == _iota2((n_from_lane, 4 * n_from_lane), 0), 1.0, 0.0)
    return w, u


def _combine(m0_ref, m1_ref, sf_ref, m2_ref, c1_ref):
    w1, u1 = _up_mats(16, 4)
    w2, u2 = _up_mats(64, 16)
    for q in range(2):
        s_c = jnp.where(m0_ref[q] != 0, 0.0, 2.0)
        s_up = jnp.dot(jnp.dot(w1, s_c), u1)
        for r in range(4):
            m1r = m1_ref[4 * q + r] != 0
            cond1 = m1r & (s_up > 0.0)
            c1_ref[0, 0, 4 * q + r] = cond1
            s_mid = jnp.where(cond1, 1.0, s_up)
            sf2d = jnp.dot(jnp.dot(w2, s_mid), u2)
            sf_ref[pl.ds(16 * q + 4 * r, 4)] = jnp.broadcast_to(
                sf2d[None], (4, 256, 64))
            m2_ref[0, 0, pl.ds(16 * q + 4 * r, 4)] = jnp.broadcast_to(
                (sf2d > 1.0)[None], (4, 256, 64))


def kernel(density, selected_scale_index, selected_grain_ratio):
    del selected_scale_index
    density = density.astype(jnp.float32)
    ks = jnp.floor(
        selected_grain_ratio[:2].astype(jnp.float32)
        * jnp.array([1024.0, 65536.0], jnp.float32)).astype(jnp.int32)

    d_mid = pl.pallas_call(
        _pool_fine_to_mid,
        grid=(8,),
        in_specs=[pl.BlockSpec((32, 256, 64), lambda g: (g, 0, 0))],
        out_specs=pl.BlockSpec((8, 64, 16), lambda g: (g, 0, 0)),
        out_shape=jax.ShapeDtypeStruct((64, 64, 16), jnp.float32),
        compiler_params=pltpu.CompilerParams(
            dimension_semantics=("parallel",)),
    )(density)
    d_fine = density

    d_coarse, m0, m1 = pl.pallas_call(
        _select,
        in_specs=[
            pl.BlockSpec(memory_space=pltpu.SMEM),
            pl.BlockSpec((64, 64, 16), lambda: (0, 0, 0)),
        ],
        out_specs=[
            pl.BlockSpec((16, 16, 4), lambda: (0, 0, 0)),
            pl.BlockSpec((16, 16, 4), lambda: (0, 0, 0)),
            pl.BlockSpec((64, 64, 16), lambda: (0, 0, 0)),
        ],
        out_shape=[
            jax.ShapeDtypeStruct((16, 16, 4), jnp.float32),
            jax.ShapeDtypeStruct((16, 16, 4), jnp.int32),
            jax.ShapeDtypeStruct((64, 64, 16), jnp.int32),
        ],
    )(ks, d_mid)

    s_fine, m2, c1 = pl.pallas_call(
        _combine,
        grid=(8,),
        in_specs=[
            pl.BlockSpec((2, 16, 4), lambda g: (g, 0, 0)),
            pl.BlockSpec((8, 64, 16), lambda g: (g, 0, 0)),
        ],
        out_specs=[
            pl.BlockSpec((32, 256, 64), lambda g: (g, 0, 0)),
            pl.BlockSpec((1, 1, 32, 256, 64), lambda g: (0, 0, g, 0, 0)),
            pl.BlockSpec((1, 1, 8, 64, 16), lambda g: (0, 0, g, 0, 0)),
        ],
        out_shape=[
            jax.ShapeDtypeStruct((256, 256, 64), jnp.float32),
            jax.ShapeDtypeStruct((1, 1, 256, 256, 64), jnp.bool_),
            jax.ShapeDtypeStruct((1, 1, 64, 64, 16), jnp.bool_),
        ],
        compiler_params=pltpu.CompilerParams(
            dimension_semantics=("parallel",)),
    )(m0, m1)

    return (
        s_fine,
        m2,
        c1,
        (m0 != 0)[None, None],
        d_coarse[None, None],
        d_mid[None, None],
        d_fine[None, None],
    )
